# Initial kernel scaffold; baseline (speedup 1.0000x reference)
import jax
import jax.numpy as jnp
from jax.experimental import pallas as pl


def kernel(values, lengths, weights, mch_sorted_ids_0, mch_sorted_ids_1, mch_sorted_ids_2, mch_sorted_ids_3):
    raise NotImplementedError("write your pallas kernel here")


# speedup vs baseline: 13.9867x; 13.9867x over previous
import functools

import jax
import jax.numpy as jnp
from jax import lax
from jax.experimental import pallas as pl
from jax.experimental.pallas import tpu as pltpu
from jax.experimental.pallas import tpu_sc as plsc

_NUM_FEATURES = 26
_BATCH = 16384
_N = _NUM_FEATURES * _BATCH
_ZCH = 250000
_ROWS = _ZCH // 16
_CPAD = 16384
_IMAX = 2147483647

_NC = 2
_NS = 16

_CHUNK = 1024
_SUB = 128
_NSUB = _CHUNK // _SUB

_SEG01 = 7 * _BATCH
_PT01 = _SEG01 // 8
_PT23 = (6 * _BATCH) // 8


def _sc_body(values_hbm, tables_hbm, coarse_hbm, out_hbm,
             coarse_v, vals_v, bidx_v, rows_v, out_v, sem):
  cid = lax.axis_index("c")
  sid = lax.axis_index("s")
  wid = sid * _NC + cid
  tid = wid >> 3
  k = wid & 7

  is01 = tid < 2
  per_tile = jnp.where(is01, _PT01, _PT23)
  seg_start = jnp.where(tid == 0, 0,
                        jnp.where(tid == 1, _SEG01,
                                  jnp.where(tid == 2, 2 * _SEG01,
                                            2 * _SEG01 + 6 * _BATCH)))
  base = seg_start + k * per_tile
  nchunks = jnp.where(is01, _PT01 // _CHUNK, _PT23 // _CHUNK)
  row_base = tid * _ROWS

  pltpu.sync_copy(coarse_hbm.at[pl.ds(tid * _CPAD, _CPAD)], coarse_v)

  def chunk_body(ci, carry):
    off = base + ci * _CHUNK
    pltpu.sync_copy(values_hbm.at[pl.ds(off, _CHUNK)], vals_v)

    for j in range(_NSUB):
      def grp1(l, carry, j=j):
        v = vals_v[pl.ds(j * _SUB + l * 16, 16)]
        pos = jnp.zeros((16,), jnp.int32)

        def step(s, pos):
          half = lax.shift_right_logical(jnp.int32(_CPAD // 2), s)
          keys = plsc.load_gather(coarse_v, [pos + half - 1])
          return jnp.where(keys < v, pos + half, pos)

        pos = lax.fori_loop(0, 14, step, pos)
        b = jnp.minimum(pos, _ROWS - 1)
        bidx_v[j, pl.ds(l * 16, 16)] = b + row_base
        return carry

      lax.fori_loop(0, _SUB // 16, grp1, 0)

    copies = [
        pltpu.async_copy(tables_hbm.at[bidx_v.at[j]], rows_v.at[j], sem)
        for j in range(_NSUB)
    ]
    for c in copies:
      c.wait()

    for j in range(_NSUB):
      def grp3(l, carry, j=j):
        voff = j * _SUB + l * 16
        v = vals_v[pl.ds(voff, 16)]
        b = bidx_v[j, pl.ds(l * 16, 16)] - row_base
        rv = l * 16 + lax.iota(jnp.int32, 16)
        pos2 = jnp.zeros((16,), jnp.int32)

        def step(s, pos2):
          half = lax.shift_right_logical(jnp.int32(8), s)
          keys = plsc.load_gather(rows_v.at[j], [rv, pos2 + half - 1])
          return jnp.where(keys < v, pos2 + half, pos2)

        pos2 = lax.fori_loop(0, 4, step, pos2)
        row_at = plsc.load_gather(rows_v.at[j], [rv, jnp.minimum(pos2, 15)])
        hit = (pos2 < 16) & (row_at == v)
        miss = lax.rem(v, jnp.full((16,), _ZCH, jnp.int32))
        out_v[pl.ds(voff, 16)] = jnp.where(hit, b * 16 + pos2, miss)
        return carry

      lax.fori_loop(0, _SUB // 16, grp3, 0)

    pltpu.sync_copy(out_v, out_hbm.at[pl.ds(off, _CHUNK)])
    return carry

  lax.fori_loop(0, nchunks, chunk_body, 0)


_sc_remap = functools.partial(
    pl.kernel,
    out_type=jax.ShapeDtypeStruct((_N,), jnp.int32),
    mesh=plsc.VectorSubcoreMesh(core_axis_name="c", subcore_axis_name="s"),
    scratch_types=[
        pltpu.VMEM((_CPAD,), jnp.int32),
        pltpu.VMEM((_CHUNK,), jnp.int32),
        pltpu.VMEM((_NSUB, _SUB), jnp.int32),
        pltpu.VMEM((_NSUB, _SUB, 16), jnp.int32),
        pltpu.VMEM((_CHUNK,), jnp.int32),
        pltpu.SemaphoreType.DMA,
    ],
    compiler_params=pltpu.CompilerParams(needs_layout_passes=False,
                                         use_tc_tiling_on_sc=False),
)(_sc_body)


def kernel(values, lengths, weights, mch_sorted_ids_0, mch_sorted_ids_1,
           mch_sorted_ids_2, mch_sorted_ids_3):
  tables = jnp.stack([mch_sorted_ids_0, mch_sorted_ids_1,
                      mch_sorted_ids_2, mch_sorted_ids_3])
  tables2d = tables.reshape(4 * _ROWS, 16)
  coarse = tables.reshape(4, _ROWS, 16)[:, :, 15]
  coarse = jnp.pad(coarse, ((0, 0), (0, _CPAD - _ROWS)),
                   constant_values=_IMAX).reshape(-1)
  out_values = _sc_remap(values, tables2d, coarse)
  return out_values, lengths, weights


# speedup vs baseline: 18.1173x; 1.2953x over previous
import functools

import jax
import jax.numpy as jnp
from jax import lax
from jax.experimental import pallas as pl
from jax.experimental.pallas import tpu as pltpu
from jax.experimental.pallas import tpu_sc as plsc

_NUM_FEATURES = 26
_BATCH = 16384
_N = _NUM_FEATURES * _BATCH
_ZCH = 250000
_ROWS = _ZCH // 16
_CPAD = 16384
_IMAX = 2147483647

_NC = 2
_NS = 16

_CHUNK = 1024
_SUB = 128
_NSUB = _CHUNK // _SUB
_ILP = 4

_SEG01 = 7 * _BATCH
_PT01 = _SEG01 // 8
_PT23 = (6 * _BATCH) // 8


def _sc_body(values_hbm, tables_hbm, coarse_hbm, out_hbm,
             coarse_v, vals_v, bidx_v, rows_v, out_v, sem):
  cid = lax.axis_index("c")
  sid = lax.axis_index("s")
  wid = sid * _NC + cid
  tid = wid >> 3
  k = wid & 7

  is01 = tid < 2
  per_tile = jnp.where(is01, _PT01, _PT23)
  seg_start = jnp.where(tid == 0, 0,
                        jnp.where(tid == 1, _SEG01,
                                  jnp.where(tid == 2, 2 * _SEG01,
                                            2 * _SEG01 + 6 * _BATCH)))
  base = seg_start + k * per_tile
  nchunks = jnp.where(is01, _PT01 // _CHUNK, _PT23 // _CHUNK)
  row_base = tid * _ROWS

  pltpu.sync_copy(coarse_hbm.at[pl.ds(tid * _CPAD, _CPAD)], coarse_v)

  def chunk_body(ci, carry):
    off = base + ci * _CHUNK
    pltpu.sync_copy(values_hbm.at[pl.ds(off, _CHUNK)], vals_v)

    def grp1(g, carry):
      voff = g * (16 * _ILP)
      v = [vals_v[pl.ds(voff + 16 * u, 16)] for u in range(_ILP)]
      p1 = [jnp.full((16,), -1, jnp.int32) for _ in range(_ILP)]
      for s in range(14):
        half = _CPAD >> (s + 1)
        probe = [p1[u] + half for u in range(_ILP)]
        keys = [plsc.load_gather(coarse_v, [probe[u]]) for u in range(_ILP)]
        p1 = [jnp.where(keys[u] < v[u], probe[u], p1[u]) for u in range(_ILP)]
      for u in range(_ILP):
        b = jnp.minimum(p1[u] + 1, _ROWS - 1)
        bidx_v[pl.ds(voff + 16 * u, 16)] = b + row_base
      return carry

    lax.fori_loop(0, _CHUNK // (16 * _ILP), grp1, 0)

    copies = [
        pltpu.async_copy(tables_hbm.at[bidx_v.at[pl.ds(j * _SUB, _SUB)]],
                         rows_v.at[pl.ds(j * _SUB, _SUB)], sem)
        for j in range(_NSUB)
    ]
    for c in copies:
      c.wait()

    def grp3(g, carry):
      voff = g * (16 * _ILP)
      iota = lax.iota(jnp.int32, 16)
      for u in range(_ILP):
        v = vals_v[pl.ds(voff + 16 * u, 16)]
        b = bidx_v[pl.ds(voff + 16 * u, 16)] - row_base
        rv = voff + 16 * u + iota
        q1 = jnp.full((16,), -1, jnp.int32)
        for s in range(4):
          half = 8 >> s
          probe = q1 + half
          keys = plsc.load_gather(rows_v, [rv, probe])
          q1 = jnp.where(keys < v, probe, q1)
        pos2 = q1 + 1
        row_at = plsc.load_gather(rows_v, [rv, jnp.minimum(pos2, 15)])
        hit = (pos2 < 16) & (row_at == v)
        miss = lax.rem(v, jnp.full((16,), _ZCH, jnp.int32))
        out_v[pl.ds(voff + 16 * u, 16)] = jnp.where(hit, b * 16 + pos2, miss)
      return carry

    lax.fori_loop(0, _CHUNK // (16 * _ILP), grp3, 0)

    pltpu.sync_copy(out_v, out_hbm.at[pl.ds(off, _CHUNK)])
    return carry

  lax.fori_loop(0, nchunks, chunk_body, 0)


_sc_remap = functools.partial(
    pl.kernel,
    out_type=jax.ShapeDtypeStruct((_N,), jnp.int32),
    mesh=plsc.VectorSubcoreMesh(core_axis_name="c", subcore_axis_name="s"),
    scratch_types=[
        pltpu.VMEM((_CPAD,), jnp.int32),
        pltpu.VMEM((_CHUNK,), jnp.int32),
        pltpu.VMEM((_CHUNK,), jnp.int32),
        pltpu.VMEM((_CHUNK, 16), jnp.int32),
        pltpu.VMEM((_CHUNK,), jnp.int32),
        pltpu.SemaphoreType.DMA,
    ],
    compiler_params=pltpu.CompilerParams(needs_layout_passes=False,
                                         use_tc_tiling_on_sc=False),
)(_sc_body)


def kernel(values, lengths, weights, mch_sorted_ids_0, mch_sorted_ids_1,
           mch_sorted_ids_2, mch_sorted_ids_3):
  tables = jnp.stack([mch_sorted_ids_0, mch_sorted_ids_1,
                      mch_sorted_ids_2, mch_sorted_ids_3])
  tables2d = tables.reshape(4 * _ROWS, 16)
  coarse = tables.reshape(4, _ROWS, 16)[:, :, 15]
  coarse = jnp.pad(coarse, ((0, 0), (0, _CPAD - _ROWS)),
                   constant_values=_IMAX).reshape(-1)
  out_values = _sc_remap(values, tables2d, coarse)
  return out_values, lengths, weights
